# baseline (speedup 1.0000x reference)
import functools

import jax
import jax.numpy as jnp
from jax.experimental import pallas as pl

N_PTS = 16384
N_VERTS = 6890
M_PAD = 6912
N_JOINTS = 24
K = 5
BLOCK_N = 384
BIG = 1e30


def _knn_combine_kernel(xa_ref, vt_ref, whi_ref, out_ref):
    xv = xa_ref[:, :]
    vt = vt_ref[:, :]
    e0 = xv[:, 0:1] - vt[0:1, :]
    e1 = xv[:, 1:2] - vt[1:2, :]
    e2 = xv[:, 2:3] - vt[2:3, :]
    d2 = e0 * e0 + e1 * e1 + e2 * e2

    b, m = d2.shape
    s_acc = jnp.zeros((b, m), dtype=jnp.float32)
    denom = jnp.zeros((b,), dtype=jnp.float32)
    for _ in range(K):
        mv = jnp.min(d2, axis=1)
        conf = jnp.exp(-jnp.minimum(mv, 4.0))
        eq = d2 == mv[:, None]
        s_acc = jnp.where(eq, conf[:, None], s_acc)
        denom = denom + conf
        d2 = jnp.where(eq, jnp.float32(BIG), d2)

    out = jnp.dot(s_acc, whi_ref[:, :], preferred_element_type=jnp.float32)
    out_ref[:, :] = out / denom[:, None]


@jax.jit
def kernel(x, smpl_tfs, smpl_verts, smpl_weights):
    del smpl_tfs
    verts = smpl_verts[0]
    w = smpl_weights[0]
    pad = M_PAD - N_VERTS
    verts_p = jnp.concatenate(
        [verts, jnp.full((pad, 3), 1.0e3, dtype=verts.dtype)], axis=0)
    w_p = jnp.concatenate(
        [w, jnp.zeros((pad, N_JOINTS), dtype=w.dtype)], axis=0)
    vt3 = verts_p.T

    n_blocks = -(-N_PTS // BLOCK_N)
    n_pad = n_blocks * BLOCK_N
    if n_pad != N_PTS:
        x = jnp.concatenate(
            [x, jnp.zeros((n_pad - N_PTS, 3), dtype=x.dtype)], axis=0)

    grid = (n_blocks,)
    out = pl.pallas_call(
        _knn_combine_kernel,
        grid=grid,
        in_specs=[
            pl.BlockSpec((BLOCK_N, 3), lambda i: (i, 0)),
            pl.BlockSpec((3, M_PAD), lambda i: (0, 0)),
            pl.BlockSpec((M_PAD, N_JOINTS), lambda i: (0, 0)),
        ],
        out_specs=pl.BlockSpec((BLOCK_N, N_JOINTS), lambda i: (i, 0)),
        out_shape=jax.ShapeDtypeStruct((n_pad, N_JOINTS), jnp.float32),
    )(x, vt3, w_p)
    return out[None, :N_PTS]


# speedup vs baseline: 1.3568x; 1.3568x over previous
import functools

import jax
import jax.numpy as jnp
from jax.experimental import pallas as pl

N_PTS = 16384
N_VERTS = 6890
M_PAD = 6912
N_JOINTS = 24
K = 5
BLOCK_N = 256
BIG = 1e30


def _knn_combine_kernel(xa_ref, vt_ref, whi_ref, out_ref):
    xv = xa_ref[:, :]
    vt = vt_ref[:, :]
    e0 = xv[:, 0:1] - vt[0:1, :]
    e1 = xv[:, 1:2] - vt[1:2, :]
    e2 = xv[:, 2:3] - vt[2:3, :]
    d2 = e0 * e0 + e1 * e1 + e2 * e2

    d2w = d2
    denom = jnp.zeros((d2.shape[0],), dtype=jnp.float32)
    for _ in range(K):
        mv = jnp.min(d2w, axis=1)
        denom = denom + jnp.exp(-jnp.minimum(mv, 4.0))
        eq = d2w == mv[:, None]
        d2w = jnp.where(eq, jnp.float32(BIG), d2w)

    flag = d2w >= jnp.float32(0.5 * BIG)
    conf_all = jnp.exp(-jnp.minimum(d2, 4.0))
    s_acc = jnp.where(flag, conf_all, 0.0)

    out = jnp.dot(s_acc, whi_ref[:, :], preferred_element_type=jnp.float32)
    out_ref[:, :] = out / denom[:, None]


@jax.jit
def kernel(x, smpl_tfs, smpl_verts, smpl_weights):
    del smpl_tfs
    verts = smpl_verts[0]
    w = smpl_weights[0]
    pad = M_PAD - N_VERTS
    verts_p = jnp.concatenate(
        [verts, jnp.full((pad, 3), 1.0e3, dtype=verts.dtype)], axis=0)
    w_p = jnp.concatenate(
        [w, jnp.zeros((pad, N_JOINTS), dtype=w.dtype)], axis=0)
    vt3 = verts_p.T

    n_blocks = -(-N_PTS // BLOCK_N)
    n_pad = n_blocks * BLOCK_N
    if n_pad != N_PTS:
        x = jnp.concatenate(
            [x, jnp.zeros((n_pad - N_PTS, 3), dtype=x.dtype)], axis=0)

    grid = (n_blocks,)
    out = pl.pallas_call(
        _knn_combine_kernel,
        grid=grid,
        in_specs=[
            pl.BlockSpec((BLOCK_N, 3), lambda i: (i, 0)),
            pl.BlockSpec((3, M_PAD), lambda i: (0, 0)),
            pl.BlockSpec((M_PAD, N_JOINTS), lambda i: (0, 0)),
        ],
        out_specs=pl.BlockSpec((BLOCK_N, N_JOINTS), lambda i: (i, 0)),
        out_shape=jax.ShapeDtypeStruct((n_pad, N_JOINTS), jnp.float32),
    )(x, vt3, w_p)
    return out[None, :N_PTS]


# speedup vs baseline: 1.3572x; 1.0002x over previous
import functools

import jax
import jax.numpy as jnp
from jax.experimental import pallas as pl

N_PTS = 16384
N_VERTS = 6890
M_PAD = 6912
N_JOINTS = 24
K = 5
BLOCK_N = 256
BIG = 1e30


def _knn_combine_kernel(xa_ref, vt_ref, whi_ref, out_ref):
    xv = xa_ref[:, :]
    vt = vt_ref[:, :]
    e0 = xv[:, 0:1] - vt[0:1, :]
    e1 = xv[:, 1:2] - vt[1:2, :]
    e2 = xv[:, 2:3] - vt[2:3, :]
    d2 = e0 * e0 + e1 * e1 + e2 * e2

    conf_all = jnp.exp(-jnp.minimum(d2, 4.0))

    d2w = d2
    denom = jnp.zeros((d2.shape[0],), dtype=jnp.float32)
    for _ in range(K):
        mv = jnp.min(d2w, axis=1)
        denom = denom + jnp.exp(-jnp.minimum(mv, 4.0))
        eq = d2w == mv[:, None]
        d2w = jnp.where(eq, jnp.float32(BIG), d2w)

    flag = d2w >= jnp.float32(0.5 * BIG)
    s_acc = jnp.where(flag, conf_all, 0.0)

    out = jnp.dot(s_acc, whi_ref[:, :], preferred_element_type=jnp.float32)
    out_ref[:, :] = out / denom[:, None]


@jax.jit
def kernel(x, smpl_tfs, smpl_verts, smpl_weights):
    del smpl_tfs
    verts = smpl_verts[0]
    w = smpl_weights[0]
    pad = M_PAD - N_VERTS
    verts_p = jnp.concatenate(
        [verts, jnp.full((pad, 3), 1.0e3, dtype=verts.dtype)], axis=0)
    w_p = jnp.concatenate(
        [w, jnp.zeros((pad, N_JOINTS), dtype=w.dtype)], axis=0)
    vt3 = verts_p.T

    n_blocks = -(-N_PTS // BLOCK_N)
    n_pad = n_blocks * BLOCK_N
    if n_pad != N_PTS:
        x = jnp.concatenate(
            [x, jnp.zeros((n_pad - N_PTS, 3), dtype=x.dtype)], axis=0)

    grid = (n_blocks,)
    out = pl.pallas_call(
        _knn_combine_kernel,
        grid=grid,
        in_specs=[
            pl.BlockSpec((BLOCK_N, 3), lambda i: (i, 0)),
            pl.BlockSpec((3, M_PAD), lambda i: (0, 0)),
            pl.BlockSpec((M_PAD, N_JOINTS), lambda i: (0, 0)),
        ],
        out_specs=pl.BlockSpec((BLOCK_N, N_JOINTS), lambda i: (i, 0)),
        out_shape=jax.ShapeDtypeStruct((n_pad, N_JOINTS), jnp.float32),
    )(x, vt3, w_p)
    return out[None, :N_PTS]
